# baseline (device time: 25011 ns/iter reference)
import jax
import jax.numpy as jnp
from jax import lax
from jax.experimental import pallas as pl
from jax.experimental.pallas import tpu as pltpu

N_DEV = 4
B, SQ, SKV, HQ, DH = 2, 256, 256, 16, 64
D_MODEL = 512
H_LOC = HQ // N_DEV


def kernel(x, Wq, K_ext, V_ext, Wo):
    my = lax.axis_index("i")
    K_loc = lax.dynamic_slice_in_dim(K_ext, my * H_LOC, H_LOC, axis=2)
    V_loc = lax.dynamic_slice_in_dim(V_ext, my * H_LOC, H_LOC, axis=2)

    def body(x_ref, wq_ref, k_ref, v_ref, wo_ref, out_ref,
             partial_ref, comm_ref, ctx_ref, send_sems, recv_sems):
        my_pos = lax.axis_index("i")

        barrier = pltpu.get_barrier_semaphore()
        for d in range(1, N_DEV):
            pl.semaphore_signal(
                barrier, inc=1,
                device_id=((my_pos + d) % N_DEV,),
                device_id_type=pl.DeviceIdType.MESH,
            )
        pl.semaphore_wait(barrier, N_DEV - 1)

        xb = x_ref[...].reshape(B * SQ, D_MODEL).astype(jnp.bfloat16)
        wq = wq_ref[...].astype(jnp.bfloat16)
        q = jnp.dot(xb, wq, preferred_element_type=jnp.float32)
        q4 = q.reshape(B, SQ, H_LOC, DH).astype(jnp.bfloat16)

        NB = SQ // 64
        for b in range(B):
            for h in range(H_LOC):
                q_blk = q4[b, :, h, :].reshape(NB, 64, DH)
                k_blk = k_ref[b, :, h, :].astype(jnp.bfloat16).reshape(NB, 64, DH)
                s = lax.dot_general(
                    q_blk, k_blk, (((2,), (2,)), ((0,), (0,))),
                    preferred_element_type=jnp.float32,
                ) * 0.125
                m = jnp.max(s, axis=2, keepdims=True)
                w = jnp.exp(s - m)
                w = w / jnp.sum(w, axis=2, keepdims=True)
                v_blk = v_ref[b, :, h, :].astype(jnp.bfloat16).reshape(NB, 64, DH)
                ctx = lax.dot_general(
                    w.astype(jnp.bfloat16), v_blk, (((2,), (1,)), ((0,), (0,))),
                    preferred_element_type=jnp.float32,
                )
                ctx_ref[b, :, h * DH:(h + 1) * DH] = (
                    ctx.reshape(SQ, DH).astype(jnp.bfloat16))

        ctx2 = ctx_ref[...].reshape(B * SQ, H_LOC * DH)
        partial = jnp.dot(ctx2, wo_ref[...].astype(jnp.bfloat16),
                          preferred_element_type=jnp.float32)
        partial_ref[...] = partial.reshape(B, SQ, D_MODEL).astype(jnp.bfloat16)

        rdmas = []
        for d in range(1, N_DEV):
            rdma = pltpu.make_async_remote_copy(
                src_ref=partial_ref,
                dst_ref=comm_ref.at[d - 1],
                send_sem=send_sems.at[d - 1],
                recv_sem=recv_sems.at[d - 1],
                device_id=((my_pos + d) % N_DEV,),
                device_id_type=pl.DeviceIdType.MESH,
            )
            rdma.start()
            rdmas.append(rdma)
        for rdma in rdmas:
            rdma.wait_recv()
        out_ref[...] = (partial_ref[...].astype(jnp.float32)
                        + comm_ref[0].astype(jnp.float32)
                        + comm_ref[1].astype(jnp.float32)
                        + comm_ref[2].astype(jnp.float32))
        for rdma in rdmas:
            rdma.wait_send()

    return pl.pallas_call(
        body,
        out_shape=jax.ShapeDtypeStruct((B, SQ, D_MODEL), jnp.float32),
        in_specs=[pl.BlockSpec(memory_space=pltpu.VMEM)] * 5,
        out_specs=pl.BlockSpec(memory_space=pltpu.VMEM),
        scratch_shapes=[
            pltpu.VMEM((B, SQ, D_MODEL), jnp.bfloat16),
            pltpu.VMEM((N_DEV - 1, B, SQ, D_MODEL), jnp.bfloat16),
            pltpu.VMEM((B, SQ, H_LOC * DH), jnp.bfloat16),
            pltpu.SemaphoreType.DMA((N_DEV - 1,)),
            pltpu.SemaphoreType.DMA((N_DEV - 1,)),
        ],
        compiler_params=pltpu.CompilerParams(collective_id=0),
    )(x, Wq, K_loc, V_loc, Wo)


# device time: 24431 ns/iter; 1.0237x vs baseline; 1.0237x over previous
import jax
import jax.numpy as jnp
from jax import lax
from jax.experimental import pallas as pl
from jax.experimental.pallas import tpu as pltpu

N_DEV = 4
B, SQ, SKV, HQ, DH = 2, 256, 256, 16, 64
D_MODEL = 512
H_LOC = HQ // N_DEV
NB = SQ // 64


def kernel(x, Wq, K_ext, V_ext, Wo):
    my = lax.axis_index("i")
    xb = x.astype(jnp.bfloat16)
    wq = Wq.astype(jnp.bfloat16)
    wo = Wo.astype(jnp.bfloat16)
    k_loc = lax.dynamic_slice_in_dim(K_ext, my * H_LOC, H_LOC, axis=2
                                     ).astype(jnp.bfloat16)
    v_loc = lax.dynamic_slice_in_dim(V_ext, my * H_LOC, H_LOC, axis=2
                                     ).astype(jnp.bfloat16)

    def body(x_ref, wq_ref, k_ref, v_ref, wo_ref, out_ref,
             partial_ref, comm_ref, ctx_ref, send_sems, recv_sems):
        my_pos = lax.axis_index("i")

        barrier = pltpu.get_barrier_semaphore()
        for d in range(1, N_DEV):
            pl.semaphore_signal(
                barrier, inc=1,
                device_id=((my_pos + d) % N_DEV,),
                device_id_type=pl.DeviceIdType.MESH,
            )
        pl.semaphore_wait(barrier, N_DEV - 1)

        rdmas = []
        for b in range(B):
            q = jnp.dot(x_ref[b], wq_ref[...],
                        preferred_element_type=jnp.float32)
            q4 = q.reshape(SQ, H_LOC, DH).astype(jnp.bfloat16)
            for h in range(H_LOC):
                q_blk = q4[:, h, :].reshape(NB, 64, DH)
                k_blk = k_ref[b, :, h, :].reshape(NB, 64, DH)
                s = lax.dot_general(
                    q_blk, k_blk, (((2,), (2,)), ((0,), (0,))),
                    preferred_element_type=jnp.float32,
                ) * 0.125
                m = jnp.max(s, axis=2, keepdims=True)
                w = jnp.exp(s - m)
                w = w / jnp.sum(w, axis=2, keepdims=True)
                v_blk = v_ref[b, :, h, :].reshape(NB, 64, DH)
                ctx = lax.dot_general(
                    w.astype(jnp.bfloat16), v_blk, (((2,), (1,)), ((0,), (0,))),
                    preferred_element_type=jnp.float32,
                )
                ctx_ref[b, :, h * DH:(h + 1) * DH] = (
                    ctx.reshape(SQ, DH).astype(jnp.bfloat16))

            partial = jnp.dot(ctx_ref[b], wo_ref[...],
                              preferred_element_type=jnp.float32)
            partial_ref[b] = partial.astype(jnp.bfloat16)

            for d in range(1, N_DEV):
                rdma = pltpu.make_async_remote_copy(
                    src_ref=partial_ref.at[b],
                    dst_ref=comm_ref.at[d - 1, b],
                    send_sem=send_sems.at[d - 1, b],
                    recv_sem=recv_sems.at[d - 1, b],
                    device_id=((my_pos + d) % N_DEV,),
                    device_id_type=pl.DeviceIdType.MESH,
                )
                rdma.start()
                rdmas.append(rdma)

        for rdma in rdmas:
            rdma.wait_recv()
        out_ref[...] = (partial_ref[...].astype(jnp.float32)
                        + comm_ref[0].astype(jnp.float32)
                        + comm_ref[1].astype(jnp.float32)
                        + comm_ref[2].astype(jnp.float32))
        for rdma in rdmas:
            rdma.wait_send()

    return pl.pallas_call(
        body,
        out_shape=jax.ShapeDtypeStruct((B, SQ, D_MODEL), jnp.float32),
        in_specs=[pl.BlockSpec(memory_space=pltpu.VMEM)] * 5,
        out_specs=pl.BlockSpec(memory_space=pltpu.VMEM),
        scratch_shapes=[
            pltpu.VMEM((B, SQ, D_MODEL), jnp.bfloat16),
            pltpu.VMEM((N_DEV - 1, B, SQ, D_MODEL), jnp.bfloat16),
            pltpu.VMEM((B, SQ, H_LOC * DH), jnp.bfloat16),
            pltpu.SemaphoreType.DMA((N_DEV - 1, B)),
            pltpu.SemaphoreType.DMA((N_DEV - 1, B)),
        ],
        compiler_params=pltpu.CompilerParams(collective_id=0),
    )(xb, wq, k_loc, v_loc, wo)
